# baseline (speedup 1.0000x reference)
import functools

import jax
import jax.numpy as jnp
from jax import lax
from jax.experimental import pallas as pl
from jax.experimental.pallas import tpu as pltpu


def _unpack_rows(mp, n_rows):
    m32 = mp.astype(jnp.int32)
    return jnp.concatenate([(m32 >> j) & 1 for j in range(8)], axis=0)


def _pack_rows(mask):
    n = mask.shape[0]
    return jnp.packbits(mask.reshape(8, n // 8, mask.shape[1]), axis=0,
                        bitorder="little")[0]


def _layer0_kernel(x_ref, w_ref, mp_ref, b_ref, h_ref):
    k = pl.program_id(0)
    nk = pl.num_programs(0)
    xb = x_ref[...].astype(jnp.bfloat16)
    mp = mp_ref[...].astype(jnp.int32)
    ng = mp.shape[0]
    for g in range(8):
        wg = jnp.where((mp << (31 - g)) < 0,
                       w_ref[pl.ds(g * ng, ng), :], 0.0).astype(jnp.bfloat16)
        pg = lax.dot_general(xb, wg, (((1,), (1,)), ((), ())),
                             preferred_element_type=jnp.float32)

        @pl.when(k == 0)
        def _():
            h_ref[:, pl.ds(g * ng, ng)] = pg

        @pl.when(k > 0)
        def _():
            h_ref[:, pl.ds(g * ng, ng)] += pg

    @pl.when(k == nk - 1)
    def _():
        h_ref[...] = jnp.maximum(h_ref[...] + b_ref[...], 0.0)


def _tail_kernel(h_ref, w1_ref, mp1_ref, b1_ref, w2_ref, mp2_ref, b2_ref,
                 o_ref):
    m1 = _unpack_rows(mp1_ref[...], w1_ref.shape[0])
    w1b = jnp.where(m1 != 0, w1_ref[...], 0.0).astype(jnp.bfloat16)
    h1 = lax.dot_general(h_ref[...].astype(jnp.bfloat16), w1b,
                         (((1,), (1,)), ((), ())),
                         preferred_element_type=jnp.float32)
    h1 = jnp.maximum(h1 + b1_ref[...], 0.0).astype(jnp.bfloat16)
    m2 = _unpack_rows(mp2_ref[...], w2_ref.shape[0])
    w2b = jnp.where(m2 != 0, w2_ref[...], 0.0).astype(jnp.bfloat16)
    out = lax.dot_general(h1, w2b, (((1,), (1,)), ((), ())),
                          preferred_element_type=jnp.float32)
    o_ref[...] = out + b2_ref[...]


@functools.partial(jax.jit, static_argnames=("block_k",))
def _masked_mlp(x, W0, b0, W1, b1, W2, b2, mask0, mask1, mask2, block_k=512):
    B, K0 = x.shape
    N0 = W0.shape[0]
    N1 = W1.shape[0]
    N2 = W2.shape[0]
    bk = min(block_k, K0)
    nk = K0 // bk

    mp0 = _pack_rows(mask0)
    mp1 = _pack_rows(mask1)
    mp2 = _pack_rows(mask2)

    h0 = pl.pallas_call(
        _layer0_kernel,
        grid=(nk,),
        in_specs=[
            pl.BlockSpec((B, bk), lambda k: (0, k)),
            pl.BlockSpec((N0, bk), lambda k: (0, k)),
            pl.BlockSpec((N0 // 8, bk), lambda k: (0, k)),
            pl.BlockSpec((1, N0), lambda k: (0, 0)),
        ],
        out_specs=pl.BlockSpec((B, N0), lambda k: (0, 0)),
        out_shape=jax.ShapeDtypeStruct((B, N0), jnp.float32),
        compiler_params=pltpu.CompilerParams(
            dimension_semantics=("arbitrary",)),
    )(x, W0, mp0, b0.reshape(1, -1))

    full = lambda *s: pl.BlockSpec(s, lambda i: tuple(0 for _ in s))
    return pl.pallas_call(
        _tail_kernel,
        grid=(1,),
        in_specs=[
            full(B, N0),
            full(N1, N0), full(N1 // 8, N0), full(1, N1),
            full(N2, N1), full(N2 // 8, N1), full(1, N2),
        ],
        out_specs=full(B, N2),
        out_shape=jax.ShapeDtypeStruct((B, N2), jnp.float32),
    )(h0, W1, mp1, b1.reshape(1, -1), W2, mp2, b2.reshape(1, -1))


def kernel(x, W0, b0, W1, b1, W2, b2, mask0, mask1, mask2):
    return _masked_mlp(x, W0, b0, W1, b1, W2, b2, mask0, mask1, mask2)


# speedup vs baseline: 1.4919x; 1.4919x over previous
import functools

import jax
import jax.numpy as jnp
from jax import lax
from jax.experimental import pallas as pl
from jax.experimental.pallas import tpu as pltpu


def _layer0_kernel(x_ref, w_ref, m_ref, b_ref, h_ref):
    k = pl.program_id(0)
    nk = pl.num_programs(0)
    xb = x_ref[...].astype(jnp.bfloat16)
    wb = jnp.where(m_ref[...] != 0, w_ref[...], 0.0).astype(jnp.bfloat16)
    part = lax.dot_general(xb, wb, (((1,), (1,)), ((), ())),
                           preferred_element_type=jnp.float32)

    @pl.when(k == 0)
    def _():
        h_ref[...] = part

    @pl.when(k > 0)
    def _():
        h_ref[...] += part

    @pl.when(k == nk - 1)
    def _():
        h_ref[...] = jnp.maximum(h_ref[...] + b_ref[...], 0.0)


def _tail_kernel(h_ref, w1_ref, m1_ref, b1_ref, w2_ref, m2_ref, b2_ref,
                 o_ref):
    w1b = jnp.where(m1_ref[...] != 0, w1_ref[...], 0.0).astype(jnp.bfloat16)
    h1 = lax.dot_general(h_ref[...].astype(jnp.bfloat16), w1b,
                         (((1,), (1,)), ((), ())),
                         preferred_element_type=jnp.float32)
    h1 = jnp.maximum(h1 + b1_ref[...], 0.0).astype(jnp.bfloat16)
    w2b = jnp.where(m2_ref[...] != 0, w2_ref[...], 0.0).astype(jnp.bfloat16)
    out = lax.dot_general(h1, w2b, (((1,), (1,)), ((), ())),
                          preferred_element_type=jnp.float32)
    o_ref[...] = out + b2_ref[...]


@functools.partial(jax.jit, static_argnames=("block_k",))
def _masked_mlp(x, W0, b0, W1, b1, W2, b2, mask0, mask1, mask2, block_k=512):
    B, K0 = x.shape
    N0 = W0.shape[0]
    N1 = W1.shape[0]
    N2 = W2.shape[0]
    bk = min(block_k, K0)
    nk = K0 // bk

    m0 = mask0.view(jnp.uint8)
    m1 = mask1.view(jnp.uint8)
    m2 = mask2.view(jnp.uint8)

    h0 = pl.pallas_call(
        _layer0_kernel,
        grid=(nk,),
        in_specs=[
            pl.BlockSpec((B, bk), lambda k: (0, k)),
            pl.BlockSpec((N0, bk), lambda k: (0, k)),
            pl.BlockSpec((N0, bk), lambda k: (0, k)),
            pl.BlockSpec((1, N0), lambda k: (0, 0)),
        ],
        out_specs=pl.BlockSpec((B, N0), lambda k: (0, 0)),
        out_shape=jax.ShapeDtypeStruct((B, N0), jnp.float32),
        compiler_params=pltpu.CompilerParams(
            dimension_semantics=("arbitrary",)),
    )(x, W0, m0, b0.reshape(1, -1))

    full = lambda *s: pl.BlockSpec(s, lambda i: tuple(0 for _ in s))
    return pl.pallas_call(
        _tail_kernel,
        grid=(1,),
        in_specs=[
            full(B, N0),
            full(N1, N0), full(N1, N0), full(1, N1),
            full(N2, N1), full(N2, N1), full(1, N2),
        ],
        out_specs=full(B, N2),
        out_shape=jax.ShapeDtypeStruct((B, N2), jnp.float32),
    )(h0, W1, m1, b1.reshape(1, -1), W2, m2, b2.reshape(1, -1))


def kernel(x, W0, b0, W1, b1, W2, b2, mask0, mask1, mask2):
    return _masked_mlp(x, W0, b0, W1, b1, W2, b2, mask0, mask1, mask2)


# speedup vs baseline: 1.7694x; 1.1860x over previous
import functools

import jax
import jax.numpy as jnp
from jax import lax
from jax.experimental import pallas as pl
from jax.experimental.pallas import tpu as pltpu


def _layer0_kernel(x_ref, w_ref, m_ref, b_ref, h_ref):
    k = pl.program_id(1)
    nk = pl.num_programs(1)
    rows = w_ref.shape[0]
    gsz = min(512, rows)

    @pl.when(k == 0)
    def _():
        h_ref[...] = jnp.zeros_like(h_ref)

    xb = x_ref[...].astype(jnp.bfloat16)
    for g in range(rows // gsz):
        sl = pl.ds(g * gsz, gsz)
        wg = jnp.where(m_ref[sl, :] != 0, w_ref[sl, :],
                       0.0).astype(jnp.bfloat16)
        pg = lax.dot_general(xb, wg, (((1,), (1,)), ((), ())),
                             preferred_element_type=jnp.float32)
        h_ref[:, sl] += pg

    @pl.when(k == nk - 1)
    def _():
        h_ref[...] = jnp.maximum(h_ref[...] + b_ref[...], 0.0)


def _tail_kernel(h_ref, w1_ref, m1_ref, b1_ref, w2_ref, m2_ref, b2_ref,
                 o_ref):
    w1b = jnp.where(m1_ref[...] != 0, w1_ref[...], 0.0).astype(jnp.bfloat16)
    h1 = lax.dot_general(h_ref[...].astype(jnp.bfloat16), w1b,
                         (((1,), (1,)), ((), ())),
                         preferred_element_type=jnp.float32)
    h1 = jnp.maximum(h1 + b1_ref[...], 0.0).astype(jnp.bfloat16)
    w2b = jnp.where(m2_ref[...] != 0, w2_ref[...], 0.0).astype(jnp.bfloat16)
    out = lax.dot_general(h1, w2b, (((1,), (1,)), ((), ())),
                          preferred_element_type=jnp.float32)
    o_ref[...] = out + b2_ref[...]


@functools.partial(jax.jit, static_argnames=("block_k",))
def _masked_mlp(x, W0, b0, W1, b1, W2, b2, mask0, mask1, mask2, block_k=1024):
    B, K0 = x.shape
    N0 = W0.shape[0]
    N1 = W1.shape[0]
    N2 = W2.shape[0]
    bk = min(block_k, K0)
    nk = K0 // bk

    m0 = mask0.view(jnp.uint8)
    m1 = mask1.view(jnp.uint8)
    m2 = mask2.view(jnp.uint8)

    nj = 2
    bj = N0 // nj
    h0 = pl.pallas_call(
        _layer0_kernel,
        grid=(nj, nk),
        in_specs=[
            pl.BlockSpec((B, bk), lambda j, k: (0, k)),
            pl.BlockSpec((bj, bk), lambda j, k: (j, k)),
            pl.BlockSpec((bj, bk), lambda j, k: (j, k)),
            pl.BlockSpec((1, bj), lambda j, k: (0, j)),
        ],
        out_specs=pl.BlockSpec((B, bj), lambda j, k: (0, j)),
        out_shape=jax.ShapeDtypeStruct((B, N0), jnp.float32),
        compiler_params=pltpu.CompilerParams(
            dimension_semantics=("arbitrary", "arbitrary")),
    )(x, W0, m0, b0.reshape(1, -1))

    full = lambda *s: pl.BlockSpec(s, lambda i: tuple(0 for _ in s))
    return pl.pallas_call(
        _tail_kernel,
        grid=(1,),
        in_specs=[
            full(B, N0),
            full(N1, N0), full(N1, N0), full(1, N1),
            full(N2, N1), full(N2, N1), full(1, N2),
        ],
        out_specs=full(B, N2),
        out_shape=jax.ShapeDtypeStruct((B, N2), jnp.float32),
    )(h0, W1, m1, b1.reshape(1, -1), W2, m2, b2.reshape(1, -1))


def kernel(x, W0, b0, W1, b1, W2, b2, mask0, mask1, mask2):
    return _masked_mlp(x, W0, b0, W1, b1, W2, b2, mask0, mask1, mask2)
